# baseline (speedup 1.0000x reference)
import functools

import jax
import jax.numpy as jnp
from jax import lax
from jax.experimental import pallas as pl
from jax.experimental.pallas import tpu as pltpu
from jax.experimental.pallas import tpu_sc as plsc

BATCH = 16384
EMB = 64
NROWS = 100000

try:
    _INFO = plsc.get_sparse_core_info()
    _NC, _NS = _INFO.num_cores, _INFO.num_subcores
except ValueError:
    _NC, _NS = 2, 16
_NW = _NC * _NS
_BPW = BATCH // _NW
_CHN = 128


def _pack_body(a, b, out):
    out[:, :EMB] = a[...]
    out[:, EMB:] = b[...]


_PBLK = 4000


def _tc_pack(a, b):
    return pl.pallas_call(
        _pack_body,
        grid=(NROWS // _PBLK,),
        in_specs=[pl.BlockSpec((_PBLK, EMB), lambda i: (i, 0)),
                  pl.BlockSpec((_PBLK, EMB), lambda i: (i, 0))],
        out_specs=pl.BlockSpec((_PBLK, 2 * EMB), lambda i: (i, 0)),
        out_shape=jax.ShapeDtypeStruct((NROWS, 2 * EMB), jnp.float32),
    )(a, b)


def _sc_gather(user_ids, item_ids, u_cat, i_cat):
    mesh = plsc.VectorSubcoreMesh(core_axis_name="c", subcore_axis_name="s")

    @functools.partial(
        pl.kernel,
        out_type=[jax.ShapeDtypeStruct((BATCH, EMB), jnp.float32),
                  jax.ShapeDtypeStruct((BATCH, 2 * EMB), jnp.float32)],
        mesh=mesh,
        scratch_types=[
            pltpu.VMEM((_BPW,), jnp.int32),
            pltpu.VMEM((_BPW,), jnp.int32),
            pltpu.VMEM((_CHN, 2 * EMB), jnp.float32),
            pltpu.VMEM((_CHN, 2 * EMB), jnp.float32),
            pltpu.VMEM((_CHN, EMB), jnp.float32),
            pltpu.VMEM((_CHN, 2 * EMB), jnp.float32),
            pltpu.SemaphoreType.DMA,
        ],
    )
    def k(uids, iids, ucat, icat, out_gmf, out_mlp,
          uidx, iidx, ubuf, ibuf, gbuf, mbuf, sem):
        wid = lax.axis_index("s") * _NC + lax.axis_index("c")
        base = wid * _BPW
        pltpu.sync_copy(uids.at[pl.ds(base, _BPW)], uidx)
        pltpu.sync_copy(iids.at[pl.ds(base, _BPW)], iidx)

        for h in range(_BPW // _CHN):
            off = h * _CHN
            rows = pl.ds(base + off, _CHN)
            pltpu.async_copy(ucat.at[uidx.at[pl.ds(off, _CHN)]], ubuf, sem)
            pltpu.async_copy(icat.at[iidx.at[pl.ds(off, _CHN)]], ibuf, sem)
            pltpu.make_async_copy(ucat.at[pl.ds(0, _CHN)], ubuf, sem).wait()
            pltpu.make_async_copy(icat.at[pl.ds(0, _CHN)], ibuf, sem).wait()

            def mul(j, _):
                for c in range(EMB // 16):
                    s = pl.ds(c * 16, 16)
                    s2 = pl.ds(EMB + c * 16, 16)
                    gbuf[j, s] = ubuf[j, s] * ibuf[j, s]
                    mbuf[j, s] = ubuf[j, s2]
                    mbuf[j, s2] = ibuf[j, s2]
                return _

            lax.fori_loop(0, _CHN, mul, 0)
            pltpu.sync_copy(gbuf, out_gmf.at[rows])
            pltpu.sync_copy(mbuf, out_mlp.at[rows])

    return k(user_ids, item_ids, u_cat, i_cat)


_BLK = 2048


def _mlp_body(gmf, mlp, w1, b1, w2, b2, w3, b3, wog, wom, bo, out):
    h = jnp.dot(mlp[...], w1[...], preferred_element_type=jnp.float32)
    h = jax.nn.relu(h + b1[...])
    h = jax.nn.relu(jnp.dot(h, w2[...], preferred_element_type=jnp.float32) + b2[...])
    h = jax.nn.relu(jnp.dot(h, w3[...], preferred_element_type=jnp.float32) + b3[...])
    o = jnp.dot(gmf[...], wog[...], preferred_element_type=jnp.float32)
    o = o + jnp.dot(h, wom[...], preferred_element_type=jnp.float32)
    out[...] = o + bo[...]


def _tc_mlp(gmf, mlp, W1, b1, W2, b2, W3, b3, Wo, bo):
    def whole(shape):
        return pl.BlockSpec(shape, lambda i: (0,) * len(shape))

    return pl.pallas_call(
        _mlp_body,
        grid=(BATCH // _BLK,),
        in_specs=[pl.BlockSpec((_BLK, EMB), lambda i: (i, 0)),
                  pl.BlockSpec((_BLK, 2 * EMB), lambda i: (i, 0)),
                  whole((2 * EMB, 128)), whole((1, 128)),
                  whole((128, 64)), whole((1, 64)),
                  whole((64, 32)), whole((1, 32)),
                  whole((EMB, 1)), whole((32, 1)), whole((1, 1))],
        out_specs=pl.BlockSpec((_BLK, 1), lambda i: (i, 0)),
        out_shape=jax.ShapeDtypeStruct((BATCH, 1), jnp.float32),
    )(gmf, mlp, W1, b1.reshape(1, -1),
      W2, b2.reshape(1, -1), W3, b3.reshape(1, -1),
      Wo[:EMB], Wo[EMB:], bo.reshape(1, -1))


def kernel(user_ids, item_ids, ue_gmf, ie_gmf, ue_mlp, ie_mlp,
           W1, b1, W2, b2, W3, b3, Wo, bo):
    user_ids = user_ids.astype(jnp.int32)
    item_ids = item_ids.astype(jnp.int32)
    u_cat = _tc_pack(ue_gmf, ue_mlp)
    i_cat = _tc_pack(ie_gmf, ie_mlp)
    gmf, mlp = _sc_gather(user_ids, item_ids, u_cat, i_cat)
    return _tc_mlp(gmf, mlp, W1, b1, W2, b2, W3, b3, Wo, bo)


# speedup vs baseline: 1.2391x; 1.2391x over previous
import functools

import jax
import jax.numpy as jnp
from jax import lax
from jax.experimental import pallas as pl
from jax.experimental.pallas import tpu as pltpu
from jax.experimental.pallas import tpu_sc as plsc

BATCH = 16384
EMB = 64

try:
    _INFO = plsc.get_sparse_core_info()
    _NC, _NS = _INFO.num_cores, _INFO.num_subcores
except ValueError:
    _NC, _NS = 2, 16
_NW = _NC * _NS
_BPW = BATCH // _NW
_CHN = 128


def _sc_gather(user_ids, item_ids, u_cat, i_cat):
    mesh = plsc.VectorSubcoreMesh(core_axis_name="c", subcore_axis_name="s")

    @functools.partial(
        pl.kernel,
        out_type=[jax.ShapeDtypeStruct((BATCH, 2 * EMB), jnp.float32)] * 2,
        mesh=mesh,
        scratch_types=[
            pltpu.VMEM((_BPW,), jnp.int32),
            pltpu.VMEM((_BPW,), jnp.int32),
            pltpu.VMEM((_CHN, 2 * EMB), jnp.float32),
            pltpu.VMEM((_CHN, 2 * EMB), jnp.float32),
            pltpu.SemaphoreType.DMA,
        ],
    )
    def k(uids, iids, ucat, icat, out_u, out_i, uidx, iidx, ubuf, ibuf, sem):
        wid = lax.axis_index("s") * _NC + lax.axis_index("c")
        base = wid * _BPW
        pltpu.sync_copy(uids.at[pl.ds(base, _BPW)], uidx)
        pltpu.sync_copy(iids.at[pl.ds(base, _BPW)], iidx)
        for h in range(_BPW // _CHN):
            off = h * _CHN
            rows = pl.ds(base + off, _CHN)
            pltpu.async_copy(ucat.at[uidx.at[pl.ds(off, _CHN)]], ubuf, sem)
            pltpu.async_copy(icat.at[iidx.at[pl.ds(off, _CHN)]], ibuf, sem)
            pltpu.make_async_copy(ucat.at[pl.ds(0, _CHN)], ubuf, sem).wait()
            pltpu.make_async_copy(icat.at[pl.ds(0, _CHN)], ibuf, sem).wait()
            pltpu.sync_copy(ubuf, out_u.at[rows])
            pltpu.sync_copy(ibuf, out_i.at[rows])

    return k(user_ids, item_ids, u_cat, i_cat)


_BLK = 2048


def _mlp_body(ur, ir, w1u, w1i, b1, w2, b2, w3, b3, wog, wom, bo, out):
    um = ur[:, EMB:]
    im = ir[:, EMB:]
    h = jnp.dot(um, w1u[...], preferred_element_type=jnp.float32)
    h = h + jnp.dot(im, w1i[...], preferred_element_type=jnp.float32)
    h = jax.nn.relu(h + b1[...])
    h = jax.nn.relu(jnp.dot(h, w2[...], preferred_element_type=jnp.float32) + b2[...])
    h = jax.nn.relu(jnp.dot(h, w3[...], preferred_element_type=jnp.float32) + b3[...])
    gmf = ur[:, :EMB] * ir[:, :EMB]
    o = jnp.dot(gmf, wog[...], preferred_element_type=jnp.float32)
    o = o + jnp.dot(h, wom[...], preferred_element_type=jnp.float32)
    out[...] = o + bo[...]


def _tc_mlp(ur, ir, W1, b1, W2, b2, W3, b3, Wo, bo):
    def whole(shape):
        return pl.BlockSpec(shape, lambda i: (0,) * len(shape))

    bspec = pl.BlockSpec((_BLK, 2 * EMB), lambda i: (i, 0))
    return pl.pallas_call(
        _mlp_body,
        grid=(BATCH // _BLK,),
        in_specs=[bspec, bspec,
                  whole((EMB, 128)), whole((EMB, 128)), whole((1, 128)),
                  whole((128, 64)), whole((1, 64)),
                  whole((64, 32)), whole((1, 32)),
                  whole((EMB, 1)), whole((32, 1)), whole((1, 1))],
        out_specs=pl.BlockSpec((_BLK, 1), lambda i: (i, 0)),
        out_shape=jax.ShapeDtypeStruct((BATCH, 1), jnp.float32),
    )(ur, ir, W1[:EMB], W1[EMB:], b1.reshape(1, -1),
      W2, b2.reshape(1, -1), W3, b3.reshape(1, -1),
      Wo[:EMB], Wo[EMB:], bo.reshape(1, -1))


def kernel(user_ids, item_ids, ue_gmf, ie_gmf, ue_mlp, ie_mlp,
           W1, b1, W2, b2, W3, b3, Wo, bo):
    user_ids = user_ids.astype(jnp.int32)
    item_ids = item_ids.astype(jnp.int32)
    u_cat = jnp.concatenate([ue_gmf, ue_mlp], axis=1)
    i_cat = jnp.concatenate([ie_gmf, ie_mlp], axis=1)
    ur, ir = _sc_gather(user_ids, item_ids, u_cat, i_cat)
    return _tc_mlp(ur, ir, W1, b1, W2, b2, W3, b3, Wo, bo)


# speedup vs baseline: 1.9078x; 1.5397x over previous
import functools

import jax
import jax.numpy as jnp
from jax import lax
from jax.experimental import pallas as pl
from jax.experimental.pallas import tpu as pltpu
from jax.experimental.pallas import tpu_sc as plsc

BATCH = 16384
EMB = 64

try:
    _INFO = plsc.get_sparse_core_info()
    _NC, _NS = _INFO.num_cores, _INFO.num_subcores
except ValueError:
    _NC, _NS = 2, 16
_NW = _NC * _NS
_BPW = BATCH // _NW
_CHN = 128


_PBLK = 2048


def _pack_body(ug_t, um_t, ig_t, im_t, out_u, out_i):
    out_u[:, :EMB] = ug_t[...].T
    out_u[:, EMB:] = um_t[...].T
    out_i[:, :EMB] = ig_t[...].T
    out_i[:, EMB:] = im_t[...].T


def _tc_pack(ue_gmf, ue_mlp, ie_gmf, ie_mlp):
    n = ue_gmf.shape[0]
    grid = (pl.cdiv(n, _PBLK),)
    tspec = pl.BlockSpec((EMB, _PBLK), lambda i: (0, i))
    ospec = pl.BlockSpec((_PBLK, 2 * EMB), lambda i: (i, 0))
    return pl.pallas_call(
        _pack_body,
        grid=grid,
        in_specs=[tspec, tspec, tspec, tspec],
        out_specs=[ospec, ospec],
        out_shape=[jax.ShapeDtypeStruct((n, 2 * EMB), jnp.float32)] * 2,
    )(ue_gmf.T, ue_mlp.T, ie_gmf.T, ie_mlp.T)


def _sc_gather(user_ids, item_ids, u_cat, i_cat):
    mesh = plsc.VectorSubcoreMesh(core_axis_name="c", subcore_axis_name="s")

    @functools.partial(
        pl.kernel,
        out_type=[jax.ShapeDtypeStruct((BATCH, 2 * EMB), jnp.float32)] * 2,
        mesh=mesh,
        scratch_types=[
            pltpu.VMEM((_BPW,), jnp.int32),
            pltpu.VMEM((_BPW,), jnp.int32),
            pltpu.VMEM((_CHN, 2 * EMB), jnp.float32),
            pltpu.VMEM((_CHN, 2 * EMB), jnp.float32),
            pltpu.SemaphoreType.DMA,
        ],
    )
    def k(uids, iids, ucat, icat, out_u, out_i, uidx, iidx, ubuf, ibuf, sem):
        wid = lax.axis_index("s") * _NC + lax.axis_index("c")
        base = wid * _BPW
        pltpu.sync_copy(uids.at[pl.ds(base, _BPW)], uidx)
        pltpu.sync_copy(iids.at[pl.ds(base, _BPW)], iidx)
        for h in range(_BPW // _CHN):
            off = h * _CHN
            rows = pl.ds(base + off, _CHN)
            pltpu.async_copy(ucat.at[uidx.at[pl.ds(off, _CHN)]], ubuf, sem)
            pltpu.async_copy(icat.at[iidx.at[pl.ds(off, _CHN)]], ibuf, sem)
            pltpu.make_async_copy(ucat.at[pl.ds(0, _CHN)], ubuf, sem).wait()
            pltpu.make_async_copy(icat.at[pl.ds(0, _CHN)], ibuf, sem).wait()
            pltpu.sync_copy(ubuf, out_u.at[rows])
            pltpu.sync_copy(ibuf, out_i.at[rows])

    return k(user_ids, item_ids, u_cat, i_cat)


_BLK = 2048


def _mlp_body(ur, ir, w1u, w1i, b1, w2, b2, w3, b3, wog, wom, bo, out):
    um = ur[:, EMB:]
    im = ir[:, EMB:]
    h = jnp.dot(um, w1u[...], preferred_element_type=jnp.float32)
    h = h + jnp.dot(im, w1i[...], preferred_element_type=jnp.float32)
    h = jax.nn.relu(h + b1[...])
    h = jax.nn.relu(jnp.dot(h, w2[...], preferred_element_type=jnp.float32) + b2[...])
    h = jax.nn.relu(jnp.dot(h, w3[...], preferred_element_type=jnp.float32) + b3[...])
    gmf = ur[:, :EMB] * ir[:, :EMB]
    o = jnp.dot(gmf, wog[...], preferred_element_type=jnp.float32)
    o = o + jnp.dot(h, wom[...], preferred_element_type=jnp.float32)
    out[...] = o + bo[...]


def _tc_mlp(ur, ir, W1, b1, W2, b2, W3, b3, Wo, bo):
    def whole(shape):
        return pl.BlockSpec(shape, lambda i: (0,) * len(shape))

    bspec = pl.BlockSpec((_BLK, 2 * EMB), lambda i: (i, 0))
    return pl.pallas_call(
        _mlp_body,
        grid=(BATCH // _BLK,),
        in_specs=[bspec, bspec,
                  whole((EMB, 128)), whole((EMB, 128)), whole((1, 128)),
                  whole((128, 64)), whole((1, 64)),
                  whole((64, 32)), whole((1, 32)),
                  whole((EMB, 1)), whole((32, 1)), whole((1, 1))],
        out_specs=pl.BlockSpec((_BLK, 1), lambda i: (i, 0)),
        out_shape=jax.ShapeDtypeStruct((BATCH, 1), jnp.float32),
    )(ur, ir, W1[:EMB], W1[EMB:], b1.reshape(1, -1),
      W2, b2.reshape(1, -1), W3, b3.reshape(1, -1),
      Wo[:EMB], Wo[EMB:], bo.reshape(1, -1))


def kernel(user_ids, item_ids, ue_gmf, ie_gmf, ue_mlp, ie_mlp,
           W1, b1, W2, b2, W3, b3, Wo, bo):
    user_ids = user_ids.astype(jnp.int32)
    item_ids = item_ids.astype(jnp.int32)
    u_cat, i_cat = _tc_pack(ue_gmf, ue_mlp, ie_gmf, ie_mlp)
    ur, ir = _sc_gather(user_ids, item_ids, u_cat, i_cat)
    return _tc_mlp(ur, ir, W1, b1, W2, b2, W3, b3, Wo, bo)


# speedup vs baseline: 1.9205x; 1.0066x over previous
import functools

import jax
import jax.numpy as jnp
from jax import lax
from jax.experimental import pallas as pl
from jax.experimental.pallas import tpu as pltpu
from jax.experimental.pallas import tpu_sc as plsc

BATCH = 16384
EMB = 64

try:
    _INFO = plsc.get_sparse_core_info()
    _NC, _NS = _INFO.num_cores, _INFO.num_subcores
except ValueError:
    _NC, _NS = 2, 16
_NW = _NC * _NS
_BPW = BATCH // _NW
_CHN = 128

_PBLK = 4096


def _pack_body(g_t, m_t, out):
    out[:, :EMB] = g_t[...].T
    out[:, EMB:] = m_t[...].T


def _tc_pack(gmf_table, mlp_table):
    n = gmf_table.shape[0]
    return pl.pallas_call(
        _pack_body,
        grid=(pl.cdiv(n, _PBLK),),
        in_specs=[pl.BlockSpec((EMB, _PBLK), lambda i: (0, i)),
                  pl.BlockSpec((EMB, _PBLK), lambda i: (0, i))],
        out_specs=pl.BlockSpec((_PBLK, 2 * EMB), lambda i: (i, 0)),
        out_shape=jax.ShapeDtypeStruct((n, 2 * EMB), jnp.float32),
    )(gmf_table.T, mlp_table.T)


def _sc_gather(ids, cat):
    mesh = plsc.VectorSubcoreMesh(core_axis_name="c", subcore_axis_name="s")

    @functools.partial(
        pl.kernel,
        out_type=jax.ShapeDtypeStruct((BATCH, 2 * EMB), jnp.float32),
        mesh=mesh,
        scratch_types=[
            pltpu.VMEM((_BPW,), jnp.int32),
            pltpu.VMEM((_CHN, 2 * EMB), jnp.float32),
            pltpu.VMEM((_CHN, 2 * EMB), jnp.float32),
            pltpu.SemaphoreType.DMA,
            pltpu.SemaphoreType.DMA,
        ],
    )
    def k(ids_ref, cat_ref, out, idx, buf0, buf1, sem0, sem1):
        wid = lax.axis_index("s") * _NC + lax.axis_index("c")
        base = wid * _BPW
        pltpu.sync_copy(ids_ref.at[pl.ds(base, _BPW)], idx)
        bufs = (buf0, buf1)
        sems = (sem0, sem1)
        nch = _BPW // _CHN
        pltpu.async_copy(cat_ref.at[idx.at[pl.ds(0, _CHN)]], buf0, sem0)
        for h in range(nch):
            if h + 1 < nch:
                pltpu.async_copy(
                    cat_ref.at[idx.at[pl.ds((h + 1) * _CHN, _CHN)]],
                    bufs[(h + 1) % 2], sems[(h + 1) % 2])
            b = bufs[h % 2]
            pltpu.make_async_copy(cat_ref.at[pl.ds(0, _CHN)], b,
                                  sems[h % 2]).wait()
            pltpu.sync_copy(b, out.at[pl.ds(base + h * _CHN, _CHN)])

    return k(ids, cat)


_BLK = 2048


def _mlp_body(ur, ir, w1u, w1i, b1, w2, b2, w3, b3, wog, wom, bo, out):
    um = ur[:, EMB:]
    im = ir[:, EMB:]
    h = jnp.dot(um, w1u[...], preferred_element_type=jnp.float32)
    h = h + jnp.dot(im, w1i[...], preferred_element_type=jnp.float32)
    h = jax.nn.relu(h + b1[...])
    h = jax.nn.relu(jnp.dot(h, w2[...], preferred_element_type=jnp.float32) + b2[...])
    h = jax.nn.relu(jnp.dot(h, w3[...], preferred_element_type=jnp.float32) + b3[...])
    gmf = ur[:, :EMB] * ir[:, :EMB]
    o = jnp.dot(gmf, wog[...], preferred_element_type=jnp.float32)
    o = o + jnp.dot(h, wom[...], preferred_element_type=jnp.float32)
    out[...] = o + bo[...]


def _tc_mlp(ur, ir, W1, b1, W2, b2, W3, b3, Wo, bo):
    def whole(shape):
        return pl.BlockSpec(shape, lambda i: (0,) * len(shape))

    bspec = pl.BlockSpec((_BLK, 2 * EMB), lambda i: (i, 0))
    return pl.pallas_call(
        _mlp_body,
        grid=(BATCH // _BLK,),
        in_specs=[bspec, bspec,
                  whole((EMB, 128)), whole((EMB, 128)), whole((1, 128)),
                  whole((128, 64)), whole((1, 64)),
                  whole((64, 32)), whole((1, 32)),
                  whole((EMB, 1)), whole((32, 1)), whole((1, 1))],
        out_specs=pl.BlockSpec((_BLK, 1), lambda i: (i, 0)),
        out_shape=jax.ShapeDtypeStruct((BATCH, 1), jnp.float32),
    )(ur, ir, W1[:EMB], W1[EMB:], b1.reshape(1, -1),
      W2, b2.reshape(1, -1), W3, b3.reshape(1, -1),
      Wo[:EMB], Wo[EMB:], bo.reshape(1, -1))


def kernel(user_ids, item_ids, ue_gmf, ie_gmf, ue_mlp, ie_mlp,
           W1, b1, W2, b2, W3, b3, Wo, bo):
    user_ids = user_ids.astype(jnp.int32)
    item_ids = item_ids.astype(jnp.int32)
    u_cat = _tc_pack(ue_gmf, ue_mlp)
    ur = _sc_gather(user_ids, u_cat)
    i_cat = _tc_pack(ie_gmf, ie_mlp)
    ir = _sc_gather(item_ids, i_cat)
    return _tc_mlp(ur, ir, W1, b1, W2, b2, W3, b3, Wo, bo)


# speedup vs baseline: 1.9366x; 1.0084x over previous
import functools

import jax
import jax.numpy as jnp
from jax import lax
from jax.experimental import pallas as pl
from jax.experimental.pallas import tpu as pltpu
from jax.experimental.pallas import tpu_sc as plsc

BATCH = 16384
EMB = 64

try:
    _INFO = plsc.get_sparse_core_info()
    _NC, _NS = _INFO.num_cores, _INFO.num_subcores
except ValueError:
    _NC, _NS = 2, 16
_NW = _NC * _NS
_BPW = BATCH // _NW
_CHN = 128

_PBLK = 4096


def _pack_body(g_t, m_t, out):
    eye = jnp.eye(EMB, dtype=jnp.float32)
    dn = (((0,), (0,)), ((), ()))
    out[:, :EMB] = lax.dot_general(g_t[...], eye, dn,
                                   preferred_element_type=jnp.float32)
    out[:, EMB:] = lax.dot_general(m_t[...], eye, dn,
                                   preferred_element_type=jnp.float32)


def _tc_pack(gmf_table, mlp_table):
    n = gmf_table.shape[0]
    return pl.pallas_call(
        _pack_body,
        grid=(pl.cdiv(n, _PBLK),),
        in_specs=[pl.BlockSpec((EMB, _PBLK), lambda i: (0, i)),
                  pl.BlockSpec((EMB, _PBLK), lambda i: (0, i))],
        out_specs=pl.BlockSpec((_PBLK, 2 * EMB), lambda i: (i, 0)),
        out_shape=jax.ShapeDtypeStruct((n, 2 * EMB), jnp.float32),
    )(gmf_table.T, mlp_table.T)


def _sc_gather(ids, cat):
    mesh = plsc.VectorSubcoreMesh(core_axis_name="c", subcore_axis_name="s")

    @functools.partial(
        pl.kernel,
        out_type=jax.ShapeDtypeStruct((BATCH, 2 * EMB), jnp.float32),
        mesh=mesh,
        scratch_types=[
            pltpu.VMEM((_BPW,), jnp.int32),
            pltpu.VMEM((_CHN, 2 * EMB), jnp.float32),
            pltpu.VMEM((_CHN, 2 * EMB), jnp.float32),
            pltpu.SemaphoreType.DMA,
            pltpu.SemaphoreType.DMA,
        ],
    )
    def k(ids_ref, cat_ref, out, idx, buf0, buf1, sem0, sem1):
        wid = lax.axis_index("s") * _NC + lax.axis_index("c")
        base = wid * _BPW
        pltpu.sync_copy(ids_ref.at[pl.ds(base, _BPW)], idx)
        bufs = (buf0, buf1)
        sems = (sem0, sem1)
        nch = _BPW // _CHN
        pltpu.async_copy(cat_ref.at[idx.at[pl.ds(0, _CHN)]], buf0, sem0)
        for h in range(nch):
            if h + 1 < nch:
                pltpu.async_copy(
                    cat_ref.at[idx.at[pl.ds((h + 1) * _CHN, _CHN)]],
                    bufs[(h + 1) % 2], sems[(h + 1) % 2])
            b = bufs[h % 2]
            pltpu.make_async_copy(cat_ref.at[pl.ds(0, _CHN)], b,
                                  sems[h % 2]).wait()
            pltpu.sync_copy(b, out.at[pl.ds(base + h * _CHN, _CHN)])

    return k(ids, cat)


_BLK = 4096


def _mlp_body(ur, ir, w1u, w1i, b1, w2, b2, w3, b3, wog, wom, bo, out):
    um = ur[:, EMB:]
    im = ir[:, EMB:]
    h = jnp.dot(um, w1u[...], preferred_element_type=jnp.float32)
    h = h + jnp.dot(im, w1i[...], preferred_element_type=jnp.float32)
    h = jax.nn.relu(h + b1[...])
    h = jax.nn.relu(jnp.dot(h, w2[...], preferred_element_type=jnp.float32) + b2[...])
    h = jax.nn.relu(jnp.dot(h, w3[...], preferred_element_type=jnp.float32) + b3[...])
    gmf = ur[:, :EMB] * ir[:, :EMB]
    o = jnp.dot(gmf, wog[...], preferred_element_type=jnp.float32)
    o = o + jnp.dot(h, wom[...], preferred_element_type=jnp.float32)
    out[...] = o + bo[...]


def _tc_mlp(ur, ir, W1, b1, W2, b2, W3, b3, Wo, bo):
    def whole(shape):
        return pl.BlockSpec(shape, lambda i: (0,) * len(shape))

    bspec = pl.BlockSpec((_BLK, 2 * EMB), lambda i: (i, 0))
    return pl.pallas_call(
        _mlp_body,
        grid=(BATCH // _BLK,),
        in_specs=[bspec, bspec,
                  whole((EMB, 128)), whole((EMB, 128)), whole((1, 128)),
                  whole((128, 64)), whole((1, 64)),
                  whole((64, 32)), whole((1, 32)),
                  whole((EMB, 1)), whole((32, 1)), whole((1, 1))],
        out_specs=pl.BlockSpec((_BLK, 1), lambda i: (i, 0)),
        out_shape=jax.ShapeDtypeStruct((BATCH, 1), jnp.float32),
    )(ur, ir, W1[:EMB], W1[EMB:], b1.reshape(1, -1),
      W2, b2.reshape(1, -1), W3, b3.reshape(1, -1),
      Wo[:EMB], Wo[EMB:], bo.reshape(1, -1))


def kernel(user_ids, item_ids, ue_gmf, ie_gmf, ue_mlp, ie_mlp,
           W1, b1, W2, b2, W3, b3, Wo, bo):
    user_ids = user_ids.astype(jnp.int32)
    item_ids = item_ids.astype(jnp.int32)
    u_cat = _tc_pack(ue_gmf, ue_mlp)
    ur = _sc_gather(user_ids, u_cat)
    i_cat = _tc_pack(ie_gmf, ie_mlp)
    ir = _sc_gather(item_ids, i_cat)
    return _tc_mlp(ur, ir, W1, b1, W2, b2, W3, b3, Wo, bo)


# speedup vs baseline: 2.0760x; 1.0720x over previous
import functools

import jax
import jax.numpy as jnp
from jax import lax
from jax.experimental import pallas as pl
from jax.experimental.pallas import tpu as pltpu
from jax.experimental.pallas import tpu_sc as plsc

BATCH = 16384
EMB = 64

try:
    _INFO = plsc.get_sparse_core_info()
    _NC, _NS = _INFO.num_cores, _INFO.num_subcores
except ValueError:
    _NC, _NS = 2, 16
_NW = _NC * _NS
_BPW = BATCH // _NW
_CHN = 128

_PBLK = 8192


def _pack_body(g_t, m_t, out):
    eye = jnp.eye(EMB, dtype=jnp.float32)
    dn = (((0,), (0,)), ((), ()))
    out[:, :EMB] = lax.dot_general(g_t[...], eye, dn,
                                   preferred_element_type=jnp.float32)
    out[:, EMB:] = lax.dot_general(m_t[...], eye, dn,
                                   preferred_element_type=jnp.float32)


def _tc_pack(gmf_table, mlp_table):
    n = gmf_table.shape[0]
    return pl.pallas_call(
        _pack_body,
        grid=(pl.cdiv(n, _PBLK),),
        in_specs=[pl.BlockSpec((EMB, _PBLK), lambda i: (0, i)),
                  pl.BlockSpec((EMB, _PBLK), lambda i: (0, i))],
        out_specs=pl.BlockSpec((_PBLK, 2 * EMB), lambda i: (i, 0)),
        out_shape=jax.ShapeDtypeStruct((n, 2 * EMB), jnp.float32),
    )(gmf_table.T, mlp_table.T)


def _sc_gather(ids, cat):
    mesh = plsc.VectorSubcoreMesh(core_axis_name="c", subcore_axis_name="s")

    @functools.partial(
        pl.kernel,
        out_type=jax.ShapeDtypeStruct((BATCH, 2 * EMB), jnp.float32),
        mesh=mesh,
        scratch_types=[
            pltpu.VMEM((_BPW,), jnp.int32),
            pltpu.VMEM((_CHN, 2 * EMB), jnp.float32),
            pltpu.VMEM((_CHN, 2 * EMB), jnp.float32),
            pltpu.SemaphoreType.DMA,
            pltpu.SemaphoreType.DMA,
        ],
    )
    def k(ids_ref, cat_ref, out, idx, buf0, buf1, sem0, sem1):
        wid = lax.axis_index("s") * _NC + lax.axis_index("c")
        base = wid * _BPW
        pltpu.sync_copy(ids_ref.at[pl.ds(base, _BPW)], idx)
        bufs = (buf0, buf1)
        sems = (sem0, sem1)
        nch = _BPW // _CHN
        pltpu.async_copy(cat_ref.at[idx.at[pl.ds(0, _CHN)]], buf0, sem0)
        for h in range(nch):
            if h + 1 < nch:
                pltpu.async_copy(
                    cat_ref.at[idx.at[pl.ds((h + 1) * _CHN, _CHN)]],
                    bufs[(h + 1) % 2], sems[(h + 1) % 2])
            b = bufs[h % 2]
            pltpu.make_async_copy(cat_ref.at[pl.ds(0, _CHN)], b,
                                  sems[h % 2]).wait()
            pltpu.sync_copy(b, out.at[pl.ds(base + h * _CHN, _CHN)])

    return k(ids, cat)


_BLK = 4096


def _mlp_body(ur, ir, w1u, w1i, b1, w2, b2, w3, b3, wog, wom, bo, out):
    um = ur[:, EMB:]
    im = ir[:, EMB:]
    h = jnp.dot(um, w1u[...], preferred_element_type=jnp.float32)
    h = h + jnp.dot(im, w1i[...], preferred_element_type=jnp.float32)
    h = jax.nn.relu(h + b1[...])
    h = jax.nn.relu(jnp.dot(h, w2[...], preferred_element_type=jnp.float32) + b2[...])
    h = jax.nn.relu(jnp.dot(h, w3[...], preferred_element_type=jnp.float32) + b3[...])
    gmf = ur[:, :EMB] * ir[:, :EMB]
    o = jnp.dot(gmf, wog[...], preferred_element_type=jnp.float32)
    o = o + jnp.dot(h, wom[...], preferred_element_type=jnp.float32)
    out[...] = o + bo[...]


def _tc_mlp(ur, ir, W1, b1, W2, b2, W3, b3, Wo, bo):
    def whole(shape):
        return pl.BlockSpec(shape, lambda i: (0,) * len(shape))

    bspec = pl.BlockSpec((_BLK, 2 * EMB), lambda i: (i, 0))
    return pl.pallas_call(
        _mlp_body,
        grid=(BATCH // _BLK,),
        in_specs=[bspec, bspec,
                  whole((EMB, 128)), whole((EMB, 128)), whole((1, 128)),
                  whole((128, 64)), whole((1, 64)),
                  whole((64, 32)), whole((1, 32)),
                  whole((EMB, 1)), whole((32, 1)), whole((1, 1))],
        out_specs=pl.BlockSpec((_BLK, 1), lambda i: (i, 0)),
        out_shape=jax.ShapeDtypeStruct((BATCH, 1), jnp.float32),
    )(ur, ir, W1[:EMB], W1[EMB:], b1.reshape(1, -1),
      W2, b2.reshape(1, -1), W3, b3.reshape(1, -1),
      Wo[:EMB], Wo[EMB:], bo.reshape(1, -1))


def kernel(user_ids, item_ids, ue_gmf, ie_gmf, ue_mlp, ie_mlp,
           W1, b1, W2, b2, W3, b3, Wo, bo):
    user_ids = user_ids.astype(jnp.int32)
    item_ids = item_ids.astype(jnp.int32)
    u_cat = _tc_pack(ue_gmf, ue_mlp)
    ur = _sc_gather(user_ids, u_cat)
    i_cat = _tc_pack(ie_gmf, ie_mlp)
    ir = _sc_gather(item_ids, i_cat)
    return _tc_mlp(ur, ir, W1, b1, W2, b2, W3, b3, Wo, bo)
